# baseline (speedup 1.0000x reference)
import functools

import jax
import jax.numpy as jnp
from jax import lax
from jax.experimental import pallas as pl
from jax.experimental.pallas import tpu as pltpu
from jax.experimental.pallas import tpu_sc as plsc

NUM_YEARS = 1000
LATENT_DIM = 64
BATCH = 16384

NC = 2
NS = 16
NW = NC * NS
B_PER_W = BATCH // NW
CHUNK = 128
N_CHUNKS = B_PER_W // CHUNK


def _gather_kernel(idx_hbm, emb_hbm, out_hbm, idx_v, rows_v, sem_g, sem_o):
    wid = lax.axis_index("s") * NC + lax.axis_index("c")
    base = wid * B_PER_W
    pltpu.sync_copy(idx_hbm.at[pl.ds(base, B_PER_W)], idx_v)
    gathers = []
    for c in range(N_CHUNKS):
        gathers.append(
            pltpu.async_copy(
                emb_hbm.at[idx_v.at[pl.ds(c * CHUNK, CHUNK)]],
                rows_v.at[pl.ds(c * CHUNK, CHUNK)],
                sem_g,
            )
        )
    writes = []
    for c in range(N_CHUNKS):
        gathers[c].wait()
        writes.append(
            pltpu.async_copy(
                rows_v.at[pl.ds(c * CHUNK, CHUNK)],
                out_hbm.at[pl.ds(base + c * CHUNK, CHUNK)],
                sem_o,
            )
        )
    for w in writes:
        w.wait()


@jax.jit
def kernel(session_year, emb):
    mesh = plsc.VectorSubcoreMesh(core_axis_name="c", subcore_axis_name="s")
    return pl.kernel(
        _gather_kernel,
        out_type=jax.ShapeDtypeStruct((BATCH, LATENT_DIM), jnp.float32),
        mesh=mesh,
        scratch_types=[
            pltpu.VMEM((B_PER_W,), jnp.int32),
            pltpu.VMEM((B_PER_W, LATENT_DIM), jnp.float32),
            pltpu.SemaphoreType.DMA,
            pltpu.SemaphoreType.DMA,
        ],
        compiler_params=pltpu.CompilerParams(use_tc_tiling_on_sc=False),
    )(session_year, emb)


# speedup vs baseline: 1.0194x; 1.0194x over previous
import functools

import jax
import jax.numpy as jnp
from jax import lax
from jax.experimental import pallas as pl
from jax.experimental.pallas import tpu as pltpu
from jax.experimental.pallas import tpu_sc as plsc

NUM_YEARS = 1000
LATENT_DIM = 64
BATCH = 16384

NC = 2
NS = 16
NW = NC * NS
B_PER_W = BATCH // NW
CHUNK = 128
N_CHUNKS = B_PER_W // CHUNK


def _gather_kernel(idx_hbm, emb_hbm, out_hbm, idx_v, rows_v, sem_g, sem_o):
    wid = lax.axis_index("s") * NC + lax.axis_index("c")
    base = wid * B_PER_W
    pltpu.sync_copy(idx_hbm.at[pl.ds(base, B_PER_W)], idx_v)
    gathers = []
    for c in range(N_CHUNKS):
        gathers.append(
            pltpu.async_copy(
                emb_hbm.at[idx_v.at[pl.ds(c * CHUNK, CHUNK)]],
                rows_v.at[pl.ds(c * CHUNK, CHUNK)],
                sem_g,
            )
        )
    half = N_CHUNKS // 2
    for c in range(half):
        gathers[c].wait()
    w0 = pltpu.async_copy(
        rows_v.at[pl.ds(0, half * CHUNK)],
        out_hbm.at[pl.ds(base, half * CHUNK)],
        sem_o,
    )
    for c in range(half, N_CHUNKS):
        gathers[c].wait()
    w1 = pltpu.async_copy(
        rows_v.at[pl.ds(half * CHUNK, B_PER_W - half * CHUNK)],
        out_hbm.at[pl.ds(base + half * CHUNK, B_PER_W - half * CHUNK)],
        sem_o,
    )
    w0.wait()
    w1.wait()


@jax.jit
def kernel(session_year, emb):
    mesh = plsc.VectorSubcoreMesh(core_axis_name="c", subcore_axis_name="s")
    return pl.kernel(
        _gather_kernel,
        out_type=jax.ShapeDtypeStruct((BATCH, LATENT_DIM), jnp.float32),
        mesh=mesh,
        scratch_types=[
            pltpu.VMEM((B_PER_W,), jnp.int32),
            pltpu.VMEM((B_PER_W, LATENT_DIM), jnp.float32),
            pltpu.SemaphoreType.DMA,
            pltpu.SemaphoreType.DMA,
        ],
        compiler_params=pltpu.CompilerParams(
            use_tc_tiling_on_sc=False,
            skip_device_barrier=True,
            disable_bounds_checks=True,
            disable_semaphore_checks=True,
        ),
    )(session_year, emb)


# speedup vs baseline: 1.0226x; 1.0032x over previous
import functools

import jax
import jax.numpy as jnp
from jax import lax
from jax.experimental import pallas as pl
from jax.experimental.pallas import tpu as pltpu
from jax.experimental.pallas import tpu_sc as plsc

NUM_YEARS = 1000
LATENT_DIM = 64
BATCH = 16384

NC = 2
NS = 16
NW = NC * NS
B_PER_W = BATCH // NW
CHUNK = 128
N_CHUNKS = B_PER_W // CHUNK


def _gather_kernel(idx_hbm, emb_hbm, out_hbm, idx_v, rows_v, sem_g, sem_o):
    wid = lax.axis_index("s") * NC + lax.axis_index("c")
    base = wid * B_PER_W
    pltpu.sync_copy(idx_hbm.at[pl.ds(base, B_PER_W)], idx_v)
    pltpu.async_copy(emb_hbm.at[idx_v], rows_v, sem_g).wait()
    pltpu.sync_copy(rows_v, out_hbm.at[pl.ds(base, B_PER_W)])
    return
    gathers = []
    for c in range(N_CHUNKS):
        gathers.append(
            pltpu.async_copy(
                emb_hbm.at[idx_v.at[pl.ds(c * CHUNK, CHUNK)]],
                rows_v.at[pl.ds(c * CHUNK, CHUNK)],
                sem_g,
            )
        )
    for g in gathers:
        g.wait()
    half = N_CHUNKS // 2
    for c in range(half):
        gathers[c].wait()
    w0 = pltpu.async_copy(
        rows_v.at[pl.ds(0, half * CHUNK)],
        out_hbm.at[pl.ds(base, half * CHUNK)],
        sem_o,
    )
    for c in range(half, N_CHUNKS):
        gathers[c].wait()
    w1 = pltpu.async_copy(
        rows_v.at[pl.ds(half * CHUNK, B_PER_W - half * CHUNK)],
        out_hbm.at[pl.ds(base + half * CHUNK, B_PER_W - half * CHUNK)],
        sem_o,
    )
    w0.wait()
    w1.wait()


@jax.jit
def kernel(session_year, emb):
    mesh = plsc.VectorSubcoreMesh(core_axis_name="c", subcore_axis_name="s")
    return pl.kernel(
        _gather_kernel,
        out_type=jax.ShapeDtypeStruct((BATCH, LATENT_DIM), jnp.float32),
        mesh=mesh,
        scratch_types=[
            pltpu.VMEM((B_PER_W,), jnp.int32),
            pltpu.VMEM((B_PER_W, LATENT_DIM), jnp.float32),
            pltpu.SemaphoreType.DMA,
            pltpu.SemaphoreType.DMA,
        ],
        compiler_params=pltpu.CompilerParams(
            use_tc_tiling_on_sc=False,
            skip_device_barrier=True,
            disable_bounds_checks=True,
            disable_semaphore_checks=True,
        ),
    )(session_year, emb)


# speedup vs baseline: 1.0927x; 1.0686x over previous
import jax
import jax.numpy as jnp
from jax import lax
from jax.experimental import pallas as pl
from jax.experimental.pallas import tpu as pltpu
from jax.experimental.pallas import tpu_sc as plsc

NUM_YEARS = 1000
LATENT_DIM = 64
BATCH = 16384

NC = 2
NS = 16
NW = NC * NS
B_PER_W = BATCH // NW


def _gather_kernel(idx_hbm, emb_hbm, out_hbm, idx_v, rows_v, table_sp, sem_g):
    sid = lax.axis_index("s")
    wid = sid * NC + lax.axis_index("c")
    base = wid * B_PER_W

    @pl.when(sid == 0)
    def _stage_table():
        pltpu.sync_copy(emb_hbm, table_sp)

    pltpu.sync_copy(idx_hbm.at[pl.ds(base, B_PER_W)], idx_v)
    plsc.subcore_barrier()
    pltpu.async_copy(table_sp.at[idx_v], rows_v, sem_g).wait()
    pltpu.sync_copy(rows_v, out_hbm.at[pl.ds(base, B_PER_W)])


@jax.jit
def kernel(session_year, emb):
    mesh = plsc.VectorSubcoreMesh(core_axis_name="c", subcore_axis_name="s")
    return pl.kernel(
        _gather_kernel,
        out_type=jax.ShapeDtypeStruct((BATCH, LATENT_DIM), jnp.float32),
        mesh=mesh,
        scratch_types=[
            pltpu.VMEM((B_PER_W,), jnp.int32),
            pltpu.VMEM((B_PER_W, LATENT_DIM), jnp.float32),
            pltpu.VMEM_SHARED((NUM_YEARS, LATENT_DIM), jnp.float32),
            pltpu.SemaphoreType.DMA,
        ],
        compiler_params=pltpu.CompilerParams(
            use_tc_tiling_on_sc=False,
            skip_device_barrier=True,
            disable_bounds_checks=True,
            disable_semaphore_checks=True,
        ),
    )(session_year, emb)


# speedup vs baseline: 1.1034x; 1.0098x over previous
import jax
import jax.numpy as jnp
from jax import lax
from jax.experimental import pallas as pl
from jax.experimental.pallas import tpu as pltpu
from jax.experimental.pallas import tpu_sc as plsc

NUM_YEARS = 1000
LATENT_DIM = 64
BATCH = 16384

NC = 2
NS = 16
NW = NC * NS
B_PER_W = BATCH // NW


def _gather_kernel(idx_hbm, emb_hbm, out_hbm, idx_v, rows_v, table_sp, sem_g, sem_o):
    sid = lax.axis_index("s")
    wid = sid * NC + lax.axis_index("c")
    base = wid * B_PER_W

    @pl.when(sid == 0)
    def _stage_table():
        pltpu.sync_copy(emb_hbm, table_sp)

    pltpu.sync_copy(idx_hbm.at[pl.ds(base, B_PER_W)], idx_v)
    plsc.subcore_barrier()
    half = B_PER_W // 2
    g0 = pltpu.async_copy(
        table_sp.at[idx_v.at[pl.ds(0, half)]], rows_v.at[pl.ds(0, half)], sem_g
    )
    g1 = pltpu.async_copy(
        table_sp.at[idx_v.at[pl.ds(half, half)]], rows_v.at[pl.ds(half, half)], sem_g
    )
    g0.wait()
    w0 = pltpu.async_copy(
        rows_v.at[pl.ds(0, half)], out_hbm.at[pl.ds(base, half)], sem_o
    )
    g1.wait()
    w1 = pltpu.async_copy(
        rows_v.at[pl.ds(half, half)], out_hbm.at[pl.ds(base + half, half)], sem_o
    )
    w0.wait()
    w1.wait()


@jax.jit
def kernel(session_year, emb):
    mesh = plsc.VectorSubcoreMesh(core_axis_name="c", subcore_axis_name="s")
    return pl.kernel(
        _gather_kernel,
        out_type=jax.ShapeDtypeStruct((BATCH, LATENT_DIM), jnp.float32),
        mesh=mesh,
        scratch_types=[
            pltpu.VMEM((B_PER_W,), jnp.int32),
            pltpu.VMEM((B_PER_W, LATENT_DIM), jnp.float32),
            pltpu.VMEM_SHARED((NUM_YEARS, LATENT_DIM), jnp.float32),
            pltpu.SemaphoreType.DMA,
            pltpu.SemaphoreType.DMA,
        ],
        compiler_params=pltpu.CompilerParams(
            use_tc_tiling_on_sc=False,
            skip_device_barrier=True,
            disable_bounds_checks=True,
            disable_semaphore_checks=True,
        ),
    )(session_year, emb)
